# baseline (speedup 1.0000x reference)
import functools

import jax
import jax.numpy as jnp
import numpy as np
from jax import lax
from jax.experimental import pallas as pl
from jax.experimental.pallas import tpu as pltpu
from jax.experimental.pallas import tpu_sc as plsc

N = 10000
D = 128
E = 320000
NC = 2
NS = 16
NW = NC * NS

NPAD = 10240
GPW = NPAD // NW
GCH = 64

EPAD = 327680
EW = EPAD // NW
ECH = 128
NCH = EW // ECH
HCH = NCH // 2
NROWS = 10112
RPT = NROWS // NS
SQRT_D = np.float32(np.sqrt(D))


def _sc_mesh():
    return plsc.VectorSubcoreMesh(core_axis_name="c", subcore_axis_name="s",
                                  num_cores=NC, num_subcores=NS)


def _embed_gather_body(ntab, ptab, nid, pid, ne_out, pe_out, idx_v, rows_v, sem):
    wid = lax.axis_index("s") * NC + lax.axis_index("c")

    def body(j, _):
        base = wid * GPW + j * GCH
        pltpu.sync_copy(nid.at[pl.ds(base, GCH)], idx_v)
        pltpu.async_copy(ntab.at[idx_v], rows_v, sem).wait()
        pltpu.sync_copy(rows_v, ne_out.at[pl.ds(base, GCH)])
        pltpu.sync_copy(pid.at[pl.ds(base, GCH)], idx_v)
        pltpu.async_copy(ptab.at[idx_v], rows_v, sem).wait()
        pltpu.sync_copy(rows_v, pe_out.at[pl.ds(base, GCH)])
        return ()

    lax.fori_loop(0, GPW // GCH, body, (), unroll=False)


def _mp_body(x, x2, srcp, dstp, zrows, out, agg_sh, sb, db, rows, gsem):
    c = lax.axis_index("c")
    s = lax.axis_index("s")
    pltpu.sync_copy(zrows, agg_sh.at[pl.ds(s * RPT, RPT)])
    wid = c * NS + s
    plsc.subcore_barrier()

    def issue_gather(slot, j):
        pltpu.async_copy(x.at[sb.at[j]], rows.at[slot], gsem.at[slot])

    def wait_gather(slot):
        pltpu.make_async_copy(x.at[pl.ds(0, ECH)], rows.at[slot],
                              gsem.at[slot]).wait()

    def scatter(slot, j):
        pltpu.sync_copy(rows.at[slot], agg_sh.at[db.at[j]], add=True)

    @pl.when(c == 0)
    def _single_core():
      for half in range(4):
        hbase = s * (4 * HCH) + half * HCH
        pltpu.sync_copy(srcp.at[pl.ds(hbase, HCH)], sb)
        pltpu.sync_copy(dstp.at[pl.ds(hbase, HCH)], db)
        issue_gather(0, 0)

        def ustep(u, _):
            j0 = u * 2
            wait_gather(0)
            issue_gather(1, j0 + 1)
            scatter(0, j0)
            wait_gather(1)

            @pl.when(u + 1 < HCH // 2)
            def _():
                issue_gather(0, j0 + 2)
            scatter(1, j0 + 1)
            return ()

        lax.fori_loop(0, HCH // 2, ustep, (), unroll=False)

    plsc.subcore_barrier()
    pltpu.sync_copy(agg_sh.at[pl.ds(s * RPT, RPT)],
                    out.at[c, pl.ds(s * RPT, RPT)])


BLK = 1000


def _ln(x, g, b):
    m = jnp.mean(x, axis=1, keepdims=True)
    v = jnp.mean((x - m) ** 2, axis=1, keepdims=True)
    return (x - m) * lax.rsqrt(v + 1e-5) * g + b


def _embed_ln_body(ne, pe, g, b, o):
    x = ne[...] * SQRT_D + pe[...]
    o[...] = _ln(x, g[...], b[...])


def _layer_body(p0, p1, enc, wl, blv, wr, g, b, o):
    agg = p0[0] + p1[0]
    x = enc[...]
    h = lax.dot_general(agg, wl[...], (((1,), (1,)), ((), ())),
                        preferred_element_type=jnp.float32)
    h = h + blv[...] + lax.dot_general(x, wr[...], (((1,), (1,)), ((), ())),
                                       preferred_element_type=jnp.float32)
    h = jnp.maximum(h, 0.0) + x
    o[...] = _ln(h, g[...], b[...])


def _row_spec():
    return pl.BlockSpec((BLK, D), lambda i: (i, 0))


def _full_spec(shape):
    return pl.BlockSpec(shape, lambda i: tuple(0 for _ in shape))


def _part_spec(core):
    return pl.BlockSpec((1, BLK, D), lambda i, core=core: (core, i, 0))


def kernel(node_emb, pos, edge, node_table, pos_table, g_emb, b_emb,
           Wl1, bl1, Wr1, g1, b1, Wl2, bl2, Wr2, g2, b2):
    i32 = jnp.int32
    f32 = jnp.float32

    nid = jnp.zeros((NPAD,), i32).at[:N].set(node_emb.astype(i32))
    pid = jnp.zeros((NPAD,), i32).at[:N].set(pos.astype(i32))
    src = jnp.zeros((EPAD,), i32).at[:E].set(edge[0].astype(i32))
    dst = jnp.full((EPAD,), N, i32).at[:E].set(edge[1].astype(i32))
    srcp = src.reshape(-1, ECH)
    dstp = dst.reshape(-1, ECH)
    zrows = jnp.zeros((RPT, D), f32)

    g_emb2, b_emb2 = g_emb.reshape(1, D), b_emb.reshape(1, D)
    bl1_2, g1_2, b1_2 = bl1.reshape(1, D), g1.reshape(1, D), b1.reshape(1, D)
    bl2_2, g2_2, b2_2 = bl2.reshape(1, D), g2.reshape(1, D), b2.reshape(1, D)

    mesh = _sc_mesh()

    embed_gather = pl.kernel(
        _embed_gather_body,
        out_type=[jax.ShapeDtypeStruct((NPAD, D), f32),
                  jax.ShapeDtypeStruct((NPAD, D), f32)],
        mesh=mesh,
        scratch_types=[
            pltpu.VMEM((GCH,), i32),
            pltpu.VMEM((GCH, D), f32),
            pltpu.SemaphoreType.DMA,
        ],
    )

    message_pass = pl.kernel(
        _mp_body,
        out_type=jax.ShapeDtypeStruct((NC, NROWS, D), f32),
        mesh=mesh,
        scratch_types=[
            pltpu.VMEM_SHARED((NROWS, D), f32),
            pltpu.VMEM((HCH, ECH), i32),
            pltpu.VMEM((HCH, ECH), i32),
            pltpu.VMEM((2, ECH, D), f32),
            pltpu.SemaphoreType.DMA((2,)),
        ],
    )

    embed_ln = pl.pallas_call(
        _embed_ln_body,
        grid=(N // BLK,),
        in_specs=[_row_spec(), _row_spec(),
                  _full_spec((1, D)), _full_spec((1, D))],
        out_specs=_row_spec(),
        out_shape=jax.ShapeDtypeStruct((N, D), f32),
    )

    def layer_tc(parts, enc, wl, blv, wr, g, b):
        return pl.pallas_call(
            _layer_body,
            grid=(N // BLK,),
            in_specs=[_part_spec(0), _part_spec(1), _row_spec(),
                      _full_spec((D, D)), _full_spec((1, D)),
                      _full_spec((D, D)), _full_spec((1, D)),
                      _full_spec((1, D))],
            out_specs=_row_spec(),
            out_shape=jax.ShapeDtypeStruct((N, D), f32),
        )(parts, parts, enc, wl, blv, wr, g, b)

    ne, pe = embed_gather(node_table, pos_table, nid, pid)
    enc = embed_ln(ne, pe, g_emb2, b_emb2)

    parts1 = message_pass(enc, jnp.pad(enc, ((0, 8), (0, 0))), srcp, dstp, zrows)
    enc = layer_tc(parts1, enc, Wl1, bl1_2, Wr1, g1_2, b1_2)

    parts2 = message_pass(enc, jnp.pad(enc, ((0, 8), (0, 0))), srcp, dstp, zrows)
    enc = layer_tc(parts2, enc, Wl2, bl2_2, Wr2, g2_2, b2_2)
    return enc


# speedup vs baseline: 2.0605x; 2.0605x over previous
import jax
import jax.numpy as jnp
import numpy as np
from jax import lax
from jax.experimental import pallas as pl
from jax.experimental.pallas import tpu as pltpu
from jax.experimental.pallas import tpu_sc as plsc

N = 10000
D = 128
E = 320000
NC = 2
NS = 16
NW = NC * NS

NPAD = 10240
GPW = NPAD // NW
GCH = 64

EPAD = 327680
C32 = 32
TOT32 = EPAD // C32
T32 = TOT32 // NS
IGRP = 8
NGRP = T32 // IGRP
NU = NGRP // 2
HALF = N // 2
NH = HALF + 8
SQRT_D = np.float32(np.sqrt(D))

XS, XSL = 624, 640
ZS, ZSL = 312, 328


def _sc_mesh():
    return plsc.VectorSubcoreMesh(core_axis_name="c", subcore_axis_name="s",
                                  num_cores=NC, num_subcores=NS)


def _embed_gather_body(ntab, ptab, nid, pid, ne_out, pe_out, idx_v, rows_v, sem):
    wid = lax.axis_index("s") * NC + lax.axis_index("c")

    def body(j, _):
        base = wid * GPW + j * GCH
        pltpu.sync_copy(nid.at[pl.ds(base, GCH)], idx_v)
        pltpu.async_copy(ntab.at[idx_v], rows_v, sem).wait()
        pltpu.sync_copy(rows_v, ne_out.at[pl.ds(base, GCH)])
        pltpu.sync_copy(pid.at[pl.ds(base, GCH)], idx_v)
        pltpu.async_copy(ptab.at[idx_v], rows_v, sem).wait()
        pltpu.sync_copy(rows_v, pe_out.at[pl.ds(base, GCH)])
        return ()

    lax.fori_loop(0, GPW // GCH, body, (), unroll=False)


def _mp_body(x, src1, dstp, zrows, out,
             x_sh, agg_sh, sb0, sb1, db0, db1, rows, gsem, isem):
    c = lax.axis_index("c")
    s = lax.axis_index("s")

    @pl.when(s < NS - 1)
    def _():
        pltpu.sync_copy(x.at[pl.ds(s * XS, XS)], x_sh.at[pl.ds(s * XS, XS)])
        pltpu.sync_copy(zrows.at[pl.ds(0, ZS)], agg_sh.at[pl.ds(s * ZS, ZS)])

    @pl.when(s == NS - 1)
    def _():
        pltpu.sync_copy(x.at[pl.ds(N - XSL, XSL)],
                        x_sh.at[pl.ds(N - XSL, XSL)])
        pltpu.sync_copy(zrows, agg_sh.at[pl.ds(NH - ZSL, ZSL)])

    plsc.subcore_barrier()

    base = s * T32
    cbase = c * HALF

    GW = IGRP * C32

    def load_idx(g, sbuf, dbuf, bsl):
        pltpu.async_copy(src1.at[pl.ds((base + g * IGRP) * C32, GW)], sbuf,
                         isem.at[bsl, 0])
        pltpu.async_copy(dstp.at[pl.ds(base + g * IGRP, IGRP)], dbuf,
                         isem.at[bsl, 1])

    def wait_idx_s(sbuf, bsl):
        pltpu.make_async_copy(src1.at[pl.ds(0, GW)], sbuf,
                              isem.at[bsl, 0]).wait()

    def wait_idx_d(dbuf, bsl):
        pltpu.make_async_copy(dstp.at[pl.ds(0, IGRP)], dbuf,
                              isem.at[bsl, 1]).wait()

    def mask_dst(dbuf):
        for k in range(IGRP):
            for v in range(2):
                dv = dbuf[k, pl.ds(v * 16, 16)]
                dl = dv - cbase
                ok = (dl >= 0) & (dl < HALF)
                dbuf[k, pl.ds(v * 16, 16)] = jnp.where(ok, dl,
                                                       HALF + (dv & 7))

    def issue_gather(sbuf, k, slot):
        pltpu.async_copy(x_sh.at[sbuf.at[pl.ds(k * C32, C32)]],
                         rows.at[slot], gsem.at[slot])

    def wait_gather(slot):
        pltpu.make_async_copy(x.at[pl.ds(0, C32)], rows.at[slot],
                              gsem.at[slot]).wait()

    def scatter(dbuf, k, slot):
        pltpu.sync_copy(rows.at[slot], agg_sh.at[dbuf.at[k]], add=True)

    load_idx(0, sb0, db0, 0)
    load_idx(1, sb1, db1, 1)
    wait_idx_s(sb0, 0)
    issue_gather(sb0, 0, 0)

    def ustep(u, _):
        a = u * 2

        def run_group(sbuf, dbuf, bsl, nxt_sbuf, nxt_bsl, have_next):
            wait_idx_d(dbuf, bsl)
            mask_dst(dbuf)
            for k in range(IGRP):
                slot = k % 2
                wait_gather(slot)
                if k + 1 < IGRP:
                    issue_gather(sbuf, k + 1, 1 - slot)
                elif have_next is True:
                    wait_idx_s(nxt_sbuf, nxt_bsl)
                    issue_gather(nxt_sbuf, 0, 1 - slot)
                else:
                    @pl.when(have_next)
                    def _(slot=slot):
                        wait_idx_s(nxt_sbuf, nxt_bsl)
                        issue_gather(nxt_sbuf, 0, 1 - slot)
                scatter(dbuf, k, slot)

        run_group(sb0, db0, 0, sb1, nxt_bsl=1, have_next=True)
        @pl.when(u + 1 < NU)
        def _():
            load_idx(a + 2, sb0, db0, 0)

        run_group(sb1, db1, 1, sb0, nxt_bsl=0, have_next=u + 1 < NU)
        @pl.when(u + 1 < NU)
        def _():
            load_idx(a + 3, sb1, db1, 1)

        return ()

    lax.fori_loop(0, NU, ustep, (), unroll=False)
    plsc.subcore_barrier()

    @pl.when(s < NS - 1)
    def _():
        pltpu.sync_copy(agg_sh.at[pl.ds(s * ZS, ZS)],
                        out.at[c, pl.ds(s * ZS, ZS)])

    @pl.when(s == NS - 1)
    def _():
        pltpu.sync_copy(agg_sh.at[pl.ds(NH - ZSL, ZSL)],
                        out.at[c, pl.ds(NH - ZSL, ZSL)])


BLK = 1000
BPH = HALF // BLK


def _ln(x, g, b):
    m = jnp.mean(x, axis=1, keepdims=True)
    v = jnp.mean((x - m) ** 2, axis=1, keepdims=True)
    return (x - m) * lax.rsqrt(v + 1e-5) * g + b


def _embed_ln_body(ne, pe, g, b, o):
    x = ne[...] * SQRT_D + pe[...]
    o[...] = _ln(x, g[...], b[...])


def _layer_body(p, enc, wl, blv, wr, g, b, o):
    agg = p[0]
    x = enc[...]
    h = lax.dot_general(agg, wl[...], (((1,), (1,)), ((), ())),
                        preferred_element_type=jnp.float32)
    h = h + blv[...] + lax.dot_general(x, wr[...], (((1,), (1,)), ((), ())),
                                       preferred_element_type=jnp.float32)
    h = jnp.maximum(h, 0.0) + x
    o[...] = _ln(h, g[...], b[...])


def _row_spec():
    return pl.BlockSpec((BLK, D), lambda i: (i, 0))


def _full_spec(shape):
    return pl.BlockSpec(shape, lambda i: tuple(0 for _ in shape))


def _part_spec():
    return pl.BlockSpec((1, BLK, D), lambda i: (i // BPH, i % BPH, 0))


def kernel(node_emb, pos, edge, node_table, pos_table, g_emb, b_emb,
           Wl1, bl1, Wr1, g1, b1, Wl2, bl2, Wr2, g2, b2):
    i32 = jnp.int32
    f32 = jnp.float32

    nid = jnp.zeros((NPAD,), i32).at[:N].set(node_emb.astype(i32))
    pid = jnp.zeros((NPAD,), i32).at[:N].set(pos.astype(i32))
    src = jnp.zeros((EPAD,), i32).at[:E].set(edge[0].astype(i32))
    dst = jnp.full((EPAD,), N, i32).at[:E].set(edge[1].astype(i32))
    dstp = dst.reshape(TOT32, C32)
    zrows = jnp.zeros((ZSL, D), f32)

    g_emb2, b_emb2 = g_emb.reshape(1, D), b_emb.reshape(1, D)
    bl1_2, g1_2, b1_2 = bl1.reshape(1, D), g1.reshape(1, D), b1.reshape(1, D)
    bl2_2, g2_2, b2_2 = bl2.reshape(1, D), g2.reshape(1, D), b2.reshape(1, D)

    mesh = _sc_mesh()

    embed_gather = pl.kernel(
        _embed_gather_body,
        out_type=[jax.ShapeDtypeStruct((NPAD, D), f32),
                  jax.ShapeDtypeStruct((NPAD, D), f32)],
        mesh=mesh,
        scratch_types=[
            pltpu.VMEM((GCH,), i32),
            pltpu.VMEM((GCH, D), f32),
            pltpu.SemaphoreType.DMA,
        ],
    )

    message_pass = pl.kernel(
        _mp_body,
        out_type=jax.ShapeDtypeStruct((NC, NH, D), f32),
        mesh=mesh,
        scratch_types=[
            pltpu.VMEM_SHARED((N, D), f32),
            pltpu.VMEM_SHARED((NH, D), f32),
            pltpu.VMEM((IGRP * C32,), i32),
            pltpu.VMEM((IGRP * C32,), i32),
            pltpu.VMEM((IGRP, C32), i32),
            pltpu.VMEM((IGRP, C32), i32),
            pltpu.VMEM((2, C32, D), f32),
            pltpu.SemaphoreType.DMA((2,)),
            pltpu.SemaphoreType.DMA((2, 2)),
        ],
    )

    embed_ln = pl.pallas_call(
        _embed_ln_body,
        grid=(N // BLK,),
        in_specs=[_row_spec(), _row_spec(),
                  _full_spec((1, D)), _full_spec((1, D))],
        out_specs=_row_spec(),
        out_shape=jax.ShapeDtypeStruct((N, D), f32),
    )

    def layer_tc(parts, enc, wl, blv, wr, g, b):
        return pl.pallas_call(
            _layer_body,
            grid=(N // BLK,),
            in_specs=[_part_spec(), _row_spec(),
                      _full_spec((D, D)), _full_spec((1, D)),
                      _full_spec((D, D)), _full_spec((1, D)),
                      _full_spec((1, D))],
            out_specs=_row_spec(),
            out_shape=jax.ShapeDtypeStruct((N, D), f32),
        )(parts, enc, wl, blv, wr, g, b)

    ne, pe = embed_gather(node_table, pos_table, nid, pid)
    enc = embed_ln(ne, pe, g_emb2, b_emb2)

    parts1 = message_pass(enc, src, dstp, zrows)
    enc = layer_tc(parts1, enc, Wl1, bl1_2, Wr1, g1_2, b1_2)

    parts2 = message_pass(enc, src, dstp, zrows)
    enc = layer_tc(parts2, enc, Wl2, bl2_2, Wr2, g2_2, b2_2)
    return enc
